# baseline (speedup 1.0000x reference)
import functools

import jax
import jax.numpy as jnp
import numpy as np
from jax import lax
from jax.experimental import pallas as pl
from jax.experimental.pallas import tpu as pltpu
from jax.experimental.pallas import tpu_sc as plsc

C = 2048
E = 131072
D = 32
H = 32
K = 2
MSG_H = 64
MSG_O = 32
TAU = 0.1

_NC = 2
_NS = 16
_NW = _NC * _NS


def _make_gather(num_tables, chunk=512):
    per_w = E // _NW
    n_chunks = per_w // chunk
    n_groups = chunk // 16
    mesh = plsc.VectorSubcoreMesh(core_axis_name="c", subcore_axis_name="s")
    out_type = [jax.ShapeDtypeStruct((E, D), jnp.float32)] * (2 * num_tables)
    scratch = [
        pltpu.VMEM((C, D), jnp.float32),
        pltpu.VMEM((chunk,), jnp.int32),
        pltpu.VMEM((chunk,), jnp.int32),
        pltpu.VMEM((chunk, D), jnp.float32),
    ]

    @functools.partial(pl.kernel, out_type=out_type, mesh=mesh,
                       scratch_types=scratch,
                       compiler_params=pltpu.CompilerParams(
                           use_tc_tiling_on_sc=False,
                           needs_layout_passes=False))
    def gather_kernel(*refs):
        tables = refs[:num_tables]
        send, rec = refs[num_tables], refs[num_tables + 1]
        outs = refs[num_tables + 2:3 * num_tables + 2]
        table_v, sidx, ridx, obuf = refs[3 * num_tables + 2:3 * num_tables + 6]
        wid = lax.axis_index("s") * _NC + lax.axis_index("c")
        base = wid * per_w
        lane = lax.iota(jnp.int32, 16)

        for ti in range(num_tables):
            pltpu.sync_copy(tables[ti], table_v)

            def chunk_step(t, carry, ti=ti):
                off = base + t * chunk
                pltpu.sync_copy(send.at[pl.ds(off, chunk)], sidx)
                pltpu.sync_copy(rec.at[pl.ds(off, chunk)], ridx)
                for which, idx_ref in ((0, sidx), (1, ridx)):
                    @plsc.parallel_loop(0, n_groups, unroll=4)
                    def group_step(g, idx_ref=idx_ref):
                        rows = idx_ref[pl.ds(g * 16, 16)]
                        orow = lane + g * 16
                        for j in range(D):
                            jcol = jnp.full((16,), j, jnp.int32)
                            vals = plsc.load_gather(table_v, [rows, jcol])
                            plsc.store_scatter(obuf, [orow, jcol], vals)
                    pltpu.sync_copy(obuf, outs[2 * ti + which].at[pl.ds(off, chunk)])
                return carry

            lax.fori_loop(0, n_chunks, chunk_step, 0)

    return gather_kernel


_gather2 = _make_gather(2)
_gather1 = _make_gather(1)


def _make_graphs_scatter(ch=2048):
    n_chunks = E // ch
    n_groups = ch // 16
    rounds = C // (16 * _NW)
    mesh = plsc.VectorSubcoreMesh(core_axis_name="c", subcore_axis_name="s")
    out_type = [jax.ShapeDtypeStruct((K, C, C), jnp.float32)]
    scratch = [
        pltpu.VMEM((16, C), jnp.float32),
        pltpu.VMEM((16, C), jnp.float32),
        pltpu.VMEM((16, C), jnp.int32),
        pltpu.VMEM((ch,), jnp.int32), pltpu.VMEM((ch,), jnp.int32),
        pltpu.VMEM((ch,), jnp.int32), pltpu.VMEM((ch,), jnp.int32),
        pltpu.VMEM((ch,), jnp.float32), pltpu.VMEM((ch,), jnp.float32),
        pltpu.VMEM((ch,), jnp.float32), pltpu.VMEM((ch,), jnp.float32),
        pltpu.SemaphoreType.DMA, pltpu.SemaphoreType.DMA,
    ]

    @functools.partial(pl.kernel, out_type=out_type, mesh=mesh,
                       scratch_types=scratch,
                       compiler_params=pltpu.CompilerParams(
                           use_tc_tiling_on_sc=False,
                           needs_layout_passes=False))
    def scatter_kernel(send, rec, e0, e1, graphs,
                       reg0, reg1, tmp, sa, sb, ra, rb_,
                       va0, vb0, va1, vb1, sem_a, sem_b):
        svs = (sa, sb)
        rvs = (ra, rb_)
        v0s = (va0, vb0)
        v1s = (va1, vb1)
        sems = (sem_a, sem_b)
        wid = lax.axis_index("s") * _NC + lax.axis_index("c")
        lane = lax.iota(jnp.int32, 16)
        zero16 = jnp.zeros((16,), jnp.float32)

        def fire(c_idx, b):
            off = c_idx * ch
            pltpu.async_copy(send.at[pl.ds(off, ch)], svs[b], sems[b])
            pltpu.async_copy(rec.at[pl.ds(off, ch)], rvs[b], sems[b])
            pltpu.async_copy(e0.at[pl.ds(off, ch)], v0s[b], sems[b])
            pltpu.async_copy(e1.at[pl.ds(off, ch)], v1s[b], sems[b])

        def drain(b):
            pltpu.make_async_copy(send.at[pl.ds(0, ch)], svs[b], sems[b]).wait()
            pltpu.make_async_copy(rec.at[pl.ds(0, ch)], rvs[b], sems[b]).wait()
            pltpu.make_async_copy(e0.at[pl.ds(0, ch)], v0s[b], sems[b]).wait()
            pltpu.make_async_copy(e1.at[pl.ds(0, ch)], v1s[b], sems[b]).wait()

        for r in range(rounds):
            lo = (r * _NW + wid) * 16

            @plsc.parallel_loop(0, 16 * C // 16, unroll=8)
            def zstep(j):
                row = j >> 7
                col = (j & 127) * 16
                reg0[row, pl.ds(col, 16)] = zero16
                reg1[row, pl.ds(col, 16)] = zero16

            def process(c_rel, b, lo=lo):
                def gstep(g, carry):
                    s = svs[b][pl.ds(g * 16, 16)]
                    valid = (s >= lo) & (s < lo + 16)

                    def dowork():
                        rr = rvs[b][pl.ds(g * 16, 16)]
                        val0 = v0s[b][pl.ds(g * 16, 16)]
                        val1 = v1s[b][pl.ds(g * 16, 16)]
                        rowv = jnp.clip(s - lo, 0, 15)
                        plsc.store_scatter(tmp, [rowv, rr], lane, mask=valid)
                        rb = plsc.load_gather(tmp, [rowv, rr], mask=valid)
                        anydup = jnp.any(valid & (rb != lane))

                        def fast():
                            ok = valid & (rb == lane)
                            plsc.store_scatter(reg0, [rowv, rr], val0, mask=ok)
                            plsc.store_scatter(reg1, [rowv, rr], val1, mask=ok)

                        def slow():
                            def sstep(j, carry2):
                                mj = valid & (lane == j)
                                plsc.store_scatter(reg0, [rowv, rr], val0, mask=mj)
                                plsc.store_scatter(reg1, [rowv, rr], val1, mask=mj)
                                return carry2
                            lax.fori_loop(0, 16, sstep, 0)

                        lax.cond(anydup, slow, fast)

                    lax.cond(jnp.any(valid), dowork, lambda: None)
                    return carry

                lax.fori_loop(0, n_groups, gstep, 0)

            fire(0, 0)

            def pairstep(t, carry):
                c0 = 2 * t
                fire(c0 + 1, 1)
                drain(0)
                process(c0, 0)
                fire(jnp.minimum(c0 + 2, n_chunks - 1), 0)
                drain(1)
                process(c0 + 1, 1)
                return carry

            lax.fori_loop(0, n_chunks // 2, pairstep, 0)
            drain(0)
            pltpu.sync_copy(reg0, graphs.at[0, pl.ds(lo, 16)])
            pltpu.sync_copy(reg1, graphs.at[1, pl.ds(lo, 16)])

    return scatter_kernel


_graphs_scatter = _make_graphs_scatter()


_HI = jax.lax.Precision.DEFAULT
_BE = 4096
_NB = E // _BE


def _dot(a, b):
    return jnp.dot(a, b, precision=_HI)


def _node_mlp_kernel(x_ref, w1_ref, b1_ref, w2_ref, b2_ref, g_ref, be_ref,
                     out_ref):
    x = jnp.maximum(_dot(x_ref[...], w1_ref[...]) + b1_ref[...][None, :], 0.0)
    x = jnp.maximum(_dot(x, w2_ref[...]) + b2_ref[...][None, :], 0.0)
    mean = jnp.mean(x, axis=0, keepdims=True)
    var = jnp.mean((x - mean) * (x - mean), axis=0, keepdims=True)
    xn = (x - mean) * jax.lax.rsqrt(var + 1e-5)
    out_ref[...] = xn * g_ref[...][None, :] + be_ref[...][None, :]


def _node_mlp(x, p, name):
    return pl.pallas_call(
        _node_mlp_kernel,
        out_shape=jax.ShapeDtypeStruct((C, H), jnp.float32),
    )(x, p[name + '_w1'], p[name + '_b1'], p[name + '_w2'], p[name + '_b2'],
      p[name + '_g'], p[name + '_be'])


def _pack_psum(y):
    s1 = jnp.sum(y, axis=0, keepdims=True)
    s2 = jnp.sum(y * y, axis=0, keepdims=True)
    ps = jnp.concatenate([s1, s2], axis=0)
    ps = jnp.concatenate([ps, jnp.zeros((6, H), jnp.float32)], axis=0)
    return jnp.concatenate([ps, jnp.zeros((8, 128 - H), jnp.float32)], axis=1)


def _edge_enc2_kernel(xs_ref, xr_ref, w1a_ref, w1b_ref, b1_ref, w2_ref,
                      b2_ref, y_ref, ps_ref):
    i = pl.program_id(0)
    h = jnp.maximum(_dot(xs_ref[...], w1a_ref[...]) +
                    _dot(xr_ref[...], w1b_ref[...]) + b1_ref[...][None, :], 0.0)
    y = jnp.maximum(_dot(h, w2_ref[...]) + b2_ref[...][None, :], 0.0)
    y_ref[...] = y

    @pl.when(i == 0)
    def _():
        ps_ref[...] = jnp.zeros_like(ps_ref)

    ps_ref[...] += _pack_psum(y)


def _edge_enc4_kernel(xs_ref, xr_ref, y2_ref, a2_ref, c2_ref, w1a_ref,
                      w1b_ref, w1c_ref, b1_ref, w2_ref, b2_ref, y_ref, ps_ref):
    i = pl.program_id(0)
    skip = y2_ref[...] * a2_ref[...][None, :] + c2_ref[...][None, :]
    h = jnp.maximum(_dot(xs_ref[...], w1a_ref[...]) +
                    _dot(xr_ref[...], w1b_ref[...]) +
                    _dot(skip, w1c_ref[...]) + b1_ref[...][None, :], 0.0)
    y = jnp.maximum(_dot(h, w2_ref[...]) + b2_ref[...][None, :], 0.0)
    y_ref[...] = y

    @pl.when(i == 0)
    def _():
        ps_ref[...] = jnp.zeros_like(ps_ref)

    ps_ref[...] += _pack_psum(y)


def _edge_logits_kernel(y4_ref, a4_ref, c4_ref, fcw_ref, fcb_ref, gn_ref,
                        edges_ref, prob_ref):
    x4 = y4_ref[...] * a4_ref[...][None, :] + c4_ref[...][None, :]
    logits = _dot(x4, fcw_ref[...]) + fcb_ref[...][None, :]
    m = jnp.max(logits, axis=-1, keepdims=True)
    ex = jnp.exp(logits - m)
    prob_ref[...] = ex / jnp.sum(ex, axis=-1, keepdims=True)
    gl = (logits + gn_ref[...]) / TAU
    m2 = jnp.max(gl, axis=-1, keepdims=True)
    ex2 = jnp.exp(gl - m2)
    edges_ref[...] = ex2 / jnp.sum(ex2, axis=-1, keepdims=True)


def _edge_msg_kernel(ds_ref, dr_ref, edges_ref,
                     w1a0_ref, w1b0_ref, b10_ref, w20_ref, b20_ref,
                     w1a1_ref, w1b1_ref, b11_ref, w21_ref, b21_ref, out_ref):
    ed = edges_ref[...]
    acc = None
    for i, (w1a, w1b, b1, w2, b2) in enumerate((
            (w1a0_ref, w1b0_ref, b10_ref, w20_ref, b20_ref),
            (w1a1_ref, w1b1_ref, b11_ref, w21_ref, b21_ref))):
        m = jnp.maximum(_dot(ds_ref[...], w1a[...]) +
                        _dot(dr_ref[...], w1b[...]) + b1[...][None, :], 0.0)
        m = jnp.maximum(_dot(m, w2[...]) + b2[...][None, :], 0.0)
        m = m * ed[:, i:i + 1]
        acc = m if acc is None else acc + m
    out_ref[...] = acc


def _head_kernel(agg_ref, w1_ref, b1_ref, w2_ref, b2_ref, out_ref):
    agg = agg_ref[...] * (1.0 / C)
    pred = jnp.maximum(_dot(agg, w1_ref[...]) + b1_ref[...][None, :], 0.0)
    out_ref[...] = _dot(pred, w2_ref[...]) + b2_ref[...][None, :]


def _eblock(ncol=H):
    return pl.BlockSpec((_BE, ncol), lambda i: (i, 0))


def _wspec(shape):
    nd = len(shape)
    return pl.BlockSpec(shape, lambda i: (0,) * nd)


def _bn_affine(ps, p, name):
    s1 = ps[0, :H]
    s2 = ps[1, :H]
    mean = s1 / E
    var = s2 / E - mean * mean
    a = p[name + '_g'] * jax.lax.rsqrt(var + 1e-5)
    c = p[name + '_be'] - mean * a
    return a, c


def kernel(data, params, send_idx, rec_idx):
    p = params
    x1 = _node_mlp(data, p, 'enc1')
    xs1, xr1, ds0, dr0 = _gather2(x1, data, send_idx, rec_idx)

    w1 = p['enc2_w1']
    y2, ps2 = pl.pallas_call(
        _edge_enc2_kernel,
        grid=(_NB,),
        in_specs=[_eblock(), _eblock(), _wspec((H, H)), _wspec((H, H)),
                  _wspec((H,)), _wspec((H, H)), _wspec((H,))],
        out_specs=[_eblock(), pl.BlockSpec((8, 128), lambda i: (0, 0))],
        out_shape=[jax.ShapeDtypeStruct((E, H), jnp.float32),
                   jax.ShapeDtypeStruct((8, 128), jnp.float32)],
    )(xs1, xr1, w1[:H], w1[H:], p['enc2_b1'], p['enc2_w2'], p['enc2_b2'])
    a2, c2 = _bn_affine(ps2, p, 'enc2')

    x2n = y2 * a2[None, :] + c2[None, :]
    z = jax.ops.segment_sum(x2n, rec_idx, num_segments=C)
    x3 = _node_mlp(z / C, p, 'enc3')
    xs3, xr3 = _gather1(x3, send_idx, rec_idx)

    w1 = p['enc4_w1']
    y4, ps4 = pl.pallas_call(
        _edge_enc4_kernel,
        grid=(_NB,),
        in_specs=[_eblock(), _eblock(), _eblock(), _wspec((H,)), _wspec((H,)),
                  _wspec((H, H)), _wspec((H, H)), _wspec((H, H)),
                  _wspec((H,)), _wspec((H, H)), _wspec((H,))],
        out_specs=[_eblock(), pl.BlockSpec((8, 128), lambda i: (0, 0))],
        out_shape=[jax.ShapeDtypeStruct((E, H), jnp.float32),
                   jax.ShapeDtypeStruct((8, 128), jnp.float32)],
    )(xs3, xr3, y2, a2, c2, w1[:H], w1[H:2 * H], w1[2 * H:],
      p['enc4_b1'], p['enc4_w2'], p['enc4_b2'])
    a4, c4 = _bn_affine(ps4, p, 'enc4')

    u = jax.random.uniform(jax.random.key(42), (E, K),
                           minval=1e-6, maxval=1.0 - 1e-6)
    gn = -jnp.log(-jnp.log(u))
    edges, prob = pl.pallas_call(
        _edge_logits_kernel,
        grid=(_NB,),
        in_specs=[_eblock(), _wspec((H,)), _wspec((H,)), _wspec((H, K)),
                  _wspec((K,)), _eblock(K)],
        out_specs=[_eblock(K), _eblock(K)],
        out_shape=[jax.ShapeDtypeStruct((E, K), jnp.float32),
                   jax.ShapeDtypeStruct((E, K), jnp.float32)],
    )(y4, a4, c4, p['fc_out_w'], p['fc_out_b'], gn)

    all_msgs = pl.pallas_call(
        _edge_msg_kernel,
        grid=(_NB,),
        in_specs=[_eblock(), _eblock(), _eblock(K),
                  _wspec((D, MSG_H)), _wspec((D, MSG_H)), _wspec((MSG_H,)),
                  _wspec((MSG_H, MSG_O)), _wspec((MSG_O,)),
                  _wspec((D, MSG_H)), _wspec((D, MSG_H)), _wspec((MSG_H,)),
                  _wspec((MSG_H, MSG_O)), _wspec((MSG_O,))],
        out_specs=_eblock(MSG_O),
        out_shape=jax.ShapeDtypeStruct((E, MSG_O), jnp.float32),
    )(ds0, dr0, edges,
      p['msg1_0_w'][:D], p['msg1_0_w'][D:], p['msg1_0_b'],
      p['msg2_0_w'], p['msg2_0_b'],
      p['msg1_1_w'][:D], p['msg1_1_w'][D:], p['msg1_1_b'],
      p['msg2_1_w'], p['msg2_1_b'])

    agg_raw = jax.ops.segment_sum(all_msgs, rec_idx, num_segments=C)
    output = pl.pallas_call(
        _head_kernel,
        out_shape=jax.ShapeDtypeStruct((C, D), jnp.float32),
    )(agg_raw, p['out1_w'], p['out1_b'], p['out2_w'], p['out2_b'])

    e0 = edges[:, 0] + 0.0
    e1 = edges[:, 1] + 0.0
    graphs = _graphs_scatter(send_idx, rec_idx, e0, e1)[0]
    return graphs, output, prob


# speedup vs baseline: 1.1189x; 1.1189x over previous
import functools

import jax
import jax.numpy as jnp
import numpy as np
from jax import lax
from jax.experimental import pallas as pl
from jax.experimental.pallas import tpu as pltpu
from jax.experimental.pallas import tpu_sc as plsc

C = 2048
E = 131072
D = 32
H = 32
K = 2
MSG_H = 64
MSG_O = 32
TAU = 0.1

_NC = 2
_NS = 16
_NW = _NC * _NS


def _make_gather(num_tables, chunk=1024):
    per_w = E // _NW
    n_chunks = per_w // chunk
    n_groups = chunk // 16
    mesh = plsc.VectorSubcoreMesh(core_axis_name="c", subcore_axis_name="s")
    out_type = [jax.ShapeDtypeStruct((E, D), jnp.float32)] * (2 * num_tables)
    scratch = [
        pltpu.VMEM((C, D), jnp.float32),
        pltpu.VMEM((chunk,), jnp.int32),
        pltpu.VMEM((chunk,), jnp.int32),
        pltpu.VMEM((chunk, D), jnp.float32),
    ]

    @functools.partial(pl.kernel, out_type=out_type, mesh=mesh,
                       scratch_types=scratch,
                       compiler_params=pltpu.CompilerParams(
                           use_tc_tiling_on_sc=False,
                           needs_layout_passes=False))
    def gather_kernel(*refs):
        tables = refs[:num_tables]
        send, rec = refs[num_tables], refs[num_tables + 1]
        outs = refs[num_tables + 2:3 * num_tables + 2]
        table_v, sidx, ridx, obuf = refs[3 * num_tables + 2:3 * num_tables + 6]
        wid = lax.axis_index("s") * _NC + lax.axis_index("c")
        base = wid * per_w
        lane = lax.iota(jnp.int32, 16)

        for ti in range(num_tables):
            pltpu.sync_copy(tables[ti], table_v)

            def chunk_step(t, carry, ti=ti):
                off = base + t * chunk
                pltpu.sync_copy(send.at[pl.ds(off, chunk)], sidx)
                pltpu.sync_copy(rec.at[pl.ds(off, chunk)], ridx)
                for which, idx_ref in ((0, sidx), (1, ridx)):
                    @plsc.parallel_loop(0, n_groups, unroll=8)
                    def group_step(g, idx_ref=idx_ref):
                        rows = idx_ref[pl.ds(g * 16, 16)]
                        orow = lane + g * 16
                        for j in range(D):
                            jcol = jnp.full((16,), j, jnp.int32)
                            vals = plsc.load_gather(table_v, [rows, jcol])
                            plsc.store_scatter(obuf, [orow, jcol], vals)
                    pltpu.sync_copy(obuf, outs[2 * ti + which].at[pl.ds(off, chunk)])
                return carry

            lax.fori_loop(0, n_chunks, chunk_step, 0)

    return gather_kernel


_gather2 = _make_gather(2)
_gather1 = _make_gather(1)


def _make_graphs_scatter(ch=2048):
    n_chunks = E // ch
    n_groups = ch // 16
    rounds = C // (16 * _NW)
    mesh = plsc.VectorSubcoreMesh(core_axis_name="c", subcore_axis_name="s")
    out_type = [jax.ShapeDtypeStruct((K, C, C), jnp.float32)]
    scratch = [
        pltpu.VMEM((16, C), jnp.float32),
        pltpu.VMEM((16, C), jnp.float32),
        pltpu.VMEM((16, C), jnp.int32),
        pltpu.VMEM((ch,), jnp.int32), pltpu.VMEM((ch,), jnp.int32),
        pltpu.VMEM((ch,), jnp.int32), pltpu.VMEM((ch,), jnp.int32),
        pltpu.VMEM((ch,), jnp.float32), pltpu.VMEM((ch,), jnp.float32),
        pltpu.VMEM((ch,), jnp.float32), pltpu.VMEM((ch,), jnp.float32),
        pltpu.SemaphoreType.DMA, pltpu.SemaphoreType.DMA,
    ]

    @functools.partial(pl.kernel, out_type=out_type, mesh=mesh,
                       scratch_types=scratch,
                       compiler_params=pltpu.CompilerParams(
                           use_tc_tiling_on_sc=False,
                           needs_layout_passes=False))
    def scatter_kernel(send, rec, e0, e1, graphs,
                       reg0, reg1, tmp, sa, sb, ra, rb_,
                       va0, vb0, va1, vb1, sem_a, sem_b):
        svs = (sa, sb)
        rvs = (ra, rb_)
        v0s = (va0, vb0)
        v1s = (va1, vb1)
        sems = (sem_a, sem_b)
        wid = lax.axis_index("s") * _NC + lax.axis_index("c")
        lane = lax.iota(jnp.int32, 16)
        zero16 = jnp.zeros((16,), jnp.float32)

        def fire(c_idx, b):
            off = c_idx * ch
            pltpu.async_copy(send.at[pl.ds(off, ch)], svs[b], sems[b])
            pltpu.async_copy(rec.at[pl.ds(off, ch)], rvs[b], sems[b])
            pltpu.async_copy(e0.at[pl.ds(off, ch)], v0s[b], sems[b])
            pltpu.async_copy(e1.at[pl.ds(off, ch)], v1s[b], sems[b])

        def drain(b):
            pltpu.make_async_copy(send.at[pl.ds(0, ch)], svs[b], sems[b]).wait()
            pltpu.make_async_copy(rec.at[pl.ds(0, ch)], rvs[b], sems[b]).wait()
            pltpu.make_async_copy(e0.at[pl.ds(0, ch)], v0s[b], sems[b]).wait()
            pltpu.make_async_copy(e1.at[pl.ds(0, ch)], v1s[b], sems[b]).wait()

        for r in range(rounds):
            lo = (r * _NW + wid) * 16

            @plsc.parallel_loop(0, 16 * C // 16, unroll=8)
            def zstep(j):
                row = j >> 7
                col = (j & 127) * 16
                reg0[row, pl.ds(col, 16)] = zero16
                reg1[row, pl.ds(col, 16)] = zero16

            def process(c_rel, b, lo=lo):
                def handle16(off16):
                    s = svs[b][pl.ds(off16, 16)]
                    valid = (s >= lo) & (s < lo + 16)

                    def dowork():
                        rr = rvs[b][pl.ds(off16, 16)]
                        val0 = v0s[b][pl.ds(off16, 16)]
                        val1 = v1s[b][pl.ds(off16, 16)]
                        rowv = jnp.clip(s - lo, 0, 15)
                        plsc.store_scatter(tmp, [rowv, rr], lane, mask=valid)
                        rb = plsc.load_gather(tmp, [rowv, rr], mask=valid)
                        anydup = jnp.any(valid & (rb != lane))

                        def fast():
                            ok = valid & (rb == lane)
                            plsc.store_scatter(reg0, [rowv, rr], val0, mask=ok)
                            plsc.store_scatter(reg1, [rowv, rr], val1, mask=ok)

                        def slow():
                            def sstep(j, carry2):
                                mj = valid & (lane == j)
                                plsc.store_scatter(reg0, [rowv, rr], val0, mask=mj)
                                plsc.store_scatter(reg1, [rowv, rr], val1, mask=mj)
                                return carry2
                            lax.fori_loop(0, 16, sstep, 0)

                        lax.cond(anydup, slow, fast)

                    lax.cond(jnp.any(valid), dowork, lambda: None)

                def gstep(g, carry):
                    sA = svs[b][pl.ds(g * 32, 16)]
                    sB = svs[b][pl.ds(g * 32 + 16, 16)]
                    hitA = jnp.any((sA >= lo) & (sA < lo + 16))
                    hitB = jnp.any((sB >= lo) & (sB < lo + 16))

                    def dopair():
                        handle16(g * 32)
                        handle16(g * 32 + 16)

                    lax.cond(hitA | hitB, dopair, lambda: None)
                    return carry

                lax.fori_loop(0, n_groups // 2, gstep, 0)

            fire(0, 0)

            def pairstep(t, carry):
                c0 = 2 * t
                fire(c0 + 1, 1)
                drain(0)
                process(c0, 0)
                fire(jnp.minimum(c0 + 2, n_chunks - 1), 0)
                drain(1)
                process(c0 + 1, 1)
                return carry

            lax.fori_loop(0, n_chunks // 2, pairstep, 0)
            drain(0)
            pltpu.sync_copy(reg0, graphs.at[0, pl.ds(lo, 16)])
            pltpu.sync_copy(reg1, graphs.at[1, pl.ds(lo, 16)])

    return scatter_kernel


_graphs_scatter = _make_graphs_scatter()


_HI = jax.lax.Precision.DEFAULT
_BE = 4096
_NB = E // _BE


def _dot(a, b):
    return jnp.dot(a, b, precision=_HI)


def _node_mlp_kernel(x_ref, w1_ref, b1_ref, w2_ref, b2_ref, g_ref, be_ref,
                     out_ref):
    x = jnp.maximum(_dot(x_ref[...], w1_ref[...]) + b1_ref[...][None, :], 0.0)
    x = jnp.maximum(_dot(x, w2_ref[...]) + b2_ref[...][None, :], 0.0)
    mean = jnp.mean(x, axis=0, keepdims=True)
    var = jnp.mean((x - mean) * (x - mean), axis=0, keepdims=True)
    xn = (x - mean) * jax.lax.rsqrt(var + 1e-5)
    out_ref[...] = xn * g_ref[...][None, :] + be_ref[...][None, :]


def _node_mlp(x, p, name):
    return pl.pallas_call(
        _node_mlp_kernel,
        out_shape=jax.ShapeDtypeStruct((C, H), jnp.float32),
    )(x, p[name + '_w1'], p[name + '_b1'], p[name + '_w2'], p[name + '_b2'],
      p[name + '_g'], p[name + '_be'])


def _pack_psum(y):
    s1 = jnp.sum(y, axis=0, keepdims=True)
    s2 = jnp.sum(y * y, axis=0, keepdims=True)
    ps = jnp.concatenate([s1, s2], axis=0)
    ps = jnp.concatenate([ps, jnp.zeros((6, H), jnp.float32)], axis=0)
    return jnp.concatenate([ps, jnp.zeros((8, 128 - H), jnp.float32)], axis=1)


def _edge_enc2_kernel(xs_ref, xr_ref, w1a_ref, w1b_ref, b1_ref, w2_ref,
                      b2_ref, y_ref, ps_ref):
    i = pl.program_id(0)
    h = jnp.maximum(_dot(xs_ref[...], w1a_ref[...]) +
                    _dot(xr_ref[...], w1b_ref[...]) + b1_ref[...][None, :], 0.0)
    y = jnp.maximum(_dot(h, w2_ref[...]) + b2_ref[...][None, :], 0.0)
    y_ref[...] = y

    @pl.when(i == 0)
    def _():
        ps_ref[...] = jnp.zeros_like(ps_ref)

    ps_ref[...] += _pack_psum(y)


def _edge_enc4_kernel(xs_ref, xr_ref, y2_ref, a2_ref, c2_ref, w1a_ref,
                      w1b_ref, w1c_ref, b1_ref, w2_ref, b2_ref, y_ref, ps_ref):
    i = pl.program_id(0)
    skip = y2_ref[...] * a2_ref[...][None, :] + c2_ref[...][None, :]
    h = jnp.maximum(_dot(xs_ref[...], w1a_ref[...]) +
                    _dot(xr_ref[...], w1b_ref[...]) +
                    _dot(skip, w1c_ref[...]) + b1_ref[...][None, :], 0.0)
    y = jnp.maximum(_dot(h, w2_ref[...]) + b2_ref[...][None, :], 0.0)
    y_ref[...] = y

    @pl.when(i == 0)
    def _():
        ps_ref[...] = jnp.zeros_like(ps_ref)

    ps_ref[...] += _pack_psum(y)


def _edge_logits_kernel(y4_ref, a4_ref, c4_ref, fcw_ref, fcb_ref, gn_ref,
                        edges_ref, prob_ref):
    x4 = y4_ref[...] * a4_ref[...][None, :] + c4_ref[...][None, :]
    logits = _dot(x4, fcw_ref[...]) + fcb_ref[...][None, :]
    m = jnp.max(logits, axis=-1, keepdims=True)
    ex = jnp.exp(logits - m)
    prob_ref[...] = ex / jnp.sum(ex, axis=-1, keepdims=True)
    gl = (logits + gn_ref[...]) / TAU
    m2 = jnp.max(gl, axis=-1, keepdims=True)
    ex2 = jnp.exp(gl - m2)
    edges_ref[...] = ex2 / jnp.sum(ex2, axis=-1, keepdims=True)


def _edge_msg_kernel(ds_ref, dr_ref, edges_ref,
                     w1a0_ref, w1b0_ref, b10_ref, w20_ref, b20_ref,
                     w1a1_ref, w1b1_ref, b11_ref, w21_ref, b21_ref, out_ref):
    ed = edges_ref[...]
    acc = None
    for i, (w1a, w1b, b1, w2, b2) in enumerate((
            (w1a0_ref, w1b0_ref, b10_ref, w20_ref, b20_ref),
            (w1a1_ref, w1b1_ref, b11_ref, w21_ref, b21_ref))):
        m = jnp.maximum(_dot(ds_ref[...], w1a[...]) +
                        _dot(dr_ref[...], w1b[...]) + b1[...][None, :], 0.0)
        m = jnp.maximum(_dot(m, w2[...]) + b2[...][None, :], 0.0)
        m = m * ed[:, i:i + 1]
        acc = m if acc is None else acc + m
    out_ref[...] = acc


def _head_kernel(agg_ref, w1_ref, b1_ref, w2_ref, b2_ref, out_ref):
    agg = agg_ref[...] * (1.0 / C)
    pred = jnp.maximum(_dot(agg, w1_ref[...]) + b1_ref[...][None, :], 0.0)
    out_ref[...] = _dot(pred, w2_ref[...]) + b2_ref[...][None, :]


def _eblock(ncol=H):
    return pl.BlockSpec((_BE, ncol), lambda i: (i, 0))


def _wspec(shape):
    nd = len(shape)
    return pl.BlockSpec(shape, lambda i: (0,) * nd)


def _bn_affine(ps, p, name):
    s1 = ps[0, :H]
    s2 = ps[1, :H]
    mean = s1 / E
    var = s2 / E - mean * mean
    a = p[name + '_g'] * jax.lax.rsqrt(var + 1e-5)
    c = p[name + '_be'] - mean * a
    return a, c


def kernel(data, params, send_idx, rec_idx):
    p = params
    x1 = _node_mlp(data, p, 'enc1')
    xs1, xr1, ds0, dr0 = _gather2(x1, data, send_idx, rec_idx)

    w1 = p['enc2_w1']
    y2, ps2 = pl.pallas_call(
        _edge_enc2_kernel,
        grid=(_NB,),
        in_specs=[_eblock(), _eblock(), _wspec((H, H)), _wspec((H, H)),
                  _wspec((H,)), _wspec((H, H)), _wspec((H,))],
        out_specs=[_eblock(), pl.BlockSpec((8, 128), lambda i: (0, 0))],
        out_shape=[jax.ShapeDtypeStruct((E, H), jnp.float32),
                   jax.ShapeDtypeStruct((8, 128), jnp.float32)],
    )(xs1, xr1, w1[:H], w1[H:], p['enc2_b1'], p['enc2_w2'], p['enc2_b2'])
    a2, c2 = _bn_affine(ps2, p, 'enc2')

    x2n = y2 * a2[None, :] + c2[None, :]
    z = jax.ops.segment_sum(x2n, rec_idx, num_segments=C)
    x3 = _node_mlp(z / C, p, 'enc3')
    xs3, xr3 = _gather1(x3, send_idx, rec_idx)

    w1 = p['enc4_w1']
    y4, ps4 = pl.pallas_call(
        _edge_enc4_kernel,
        grid=(_NB,),
        in_specs=[_eblock(), _eblock(), _eblock(), _wspec((H,)), _wspec((H,)),
                  _wspec((H, H)), _wspec((H, H)), _wspec((H, H)),
                  _wspec((H,)), _wspec((H, H)), _wspec((H,))],
        out_specs=[_eblock(), pl.BlockSpec((8, 128), lambda i: (0, 0))],
        out_shape=[jax.ShapeDtypeStruct((E, H), jnp.float32),
                   jax.ShapeDtypeStruct((8, 128), jnp.float32)],
    )(xs3, xr3, y2, a2, c2, w1[:H], w1[H:2 * H], w1[2 * H:],
      p['enc4_b1'], p['enc4_w2'], p['enc4_b2'])
    a4, c4 = _bn_affine(ps4, p, 'enc4')

    u = jax.random.uniform(jax.random.key(42), (E, K),
                           minval=1e-6, maxval=1.0 - 1e-6)
    gn = -jnp.log(-jnp.log(u))
    edges, prob = pl.pallas_call(
        _edge_logits_kernel,
        grid=(_NB,),
        in_specs=[_eblock(), _wspec((H,)), _wspec((H,)), _wspec((H, K)),
                  _wspec((K,)), _eblock(K)],
        out_specs=[_eblock(K), _eblock(K)],
        out_shape=[jax.ShapeDtypeStruct((E, K), jnp.float32),
                   jax.ShapeDtypeStruct((E, K), jnp.float32)],
    )(y4, a4, c4, p['fc_out_w'], p['fc_out_b'], gn)

    all_msgs = pl.pallas_call(
        _edge_msg_kernel,
        grid=(_NB,),
        in_specs=[_eblock(), _eblock(), _eblock(K),
                  _wspec((D, MSG_H)), _wspec((D, MSG_H)), _wspec((MSG_H,)),
                  _wspec((MSG_H, MSG_O)), _wspec((MSG_O,)),
                  _wspec((D, MSG_H)), _wspec((D, MSG_H)), _wspec((MSG_H,)),
                  _wspec((MSG_H, MSG_O)), _wspec((MSG_O,))],
        out_specs=_eblock(MSG_O),
        out_shape=jax.ShapeDtypeStruct((E, MSG_O), jnp.float32),
    )(ds0, dr0, edges,
      p['msg1_0_w'][:D], p['msg1_0_w'][D:], p['msg1_0_b'],
      p['msg2_0_w'], p['msg2_0_b'],
      p['msg1_1_w'][:D], p['msg1_1_w'][D:], p['msg1_1_b'],
      p['msg2_1_w'], p['msg2_1_b'])

    agg_raw = jax.ops.segment_sum(all_msgs, rec_idx, num_segments=C)
    output = pl.pallas_call(
        _head_kernel,
        out_shape=jax.ShapeDtypeStruct((C, D), jnp.float32),
    )(agg_raw, p['out1_w'], p['out1_b'], p['out2_w'], p['out2_b'])

    e0 = edges[:, 0] + 0.0
    e1 = edges[:, 1] + 0.0
    graphs = _graphs_scatter(send_idx, rec_idx, e0, e1)[0]
    return graphs, output, prob


# speedup vs baseline: 1.1814x; 1.0558x over previous
import functools

import jax
import jax.numpy as jnp
import numpy as np
from jax import lax
from jax.experimental import pallas as pl
from jax.experimental.pallas import tpu as pltpu
from jax.experimental.pallas import tpu_sc as plsc

C = 2048
E = 131072
D = 32
H = 32
K = 2
MSG_H = 64
MSG_O = 32
TAU = 0.1

_NC = 2
_NS = 16
_NW = _NC * _NS


def _make_gather(num_tables, chunk=1024):
    per_w = E // _NW
    n_chunks = per_w // chunk
    n_groups = chunk // 16
    mesh = plsc.VectorSubcoreMesh(core_axis_name="c", subcore_axis_name="s")
    out_type = [jax.ShapeDtypeStruct((E, D), jnp.float32)] * (2 * num_tables)
    scratch = [
        pltpu.VMEM((C, D), jnp.float32),
        pltpu.VMEM((chunk,), jnp.int32),
        pltpu.VMEM((chunk,), jnp.int32),
        pltpu.VMEM((chunk, D), jnp.float32),
    ]

    @functools.partial(pl.kernel, out_type=out_type, mesh=mesh,
                       scratch_types=scratch,
                       compiler_params=pltpu.CompilerParams(
                           use_tc_tiling_on_sc=False,
                           needs_layout_passes=False))
    def gather_kernel(*refs):
        tables = refs[:num_tables]
        send, rec = refs[num_tables], refs[num_tables + 1]
        outs = refs[num_tables + 2:3 * num_tables + 2]
        table_v, sidx, ridx, obuf = refs[3 * num_tables + 2:3 * num_tables + 6]
        wid = lax.axis_index("s") * _NC + lax.axis_index("c")
        base = wid * per_w
        lane = lax.iota(jnp.int32, 16)

        for ti in range(num_tables):
            pltpu.sync_copy(tables[ti], table_v)

            def chunk_step(t, carry, ti=ti):
                off = base + t * chunk
                pltpu.sync_copy(send.at[pl.ds(off, chunk)], sidx)
                pltpu.sync_copy(rec.at[pl.ds(off, chunk)], ridx)
                for which, idx_ref in ((0, sidx), (1, ridx)):
                    @plsc.parallel_loop(0, n_groups, unroll=8)
                    def group_step(g, idx_ref=idx_ref):
                        rows = idx_ref[pl.ds(g * 16, 16)]
                        orow = lane + g * 16
                        for j in range(D):
                            jcol = jnp.full((16,), j, jnp.int32)
                            vals = plsc.load_gather(table_v, [rows, jcol])
                            plsc.store_scatter(obuf, [orow, jcol], vals)
                    pltpu.sync_copy(obuf, outs[2 * ti + which].at[pl.ds(off, chunk)])
                return carry

            lax.fori_loop(0, n_chunks, chunk_step, 0)

    return gather_kernel


_gather2 = _make_gather(2)
_gather1 = _make_gather(1)

_GN_NP = np.asarray(
    -jnp.log(-jnp.log(jax.random.uniform(
        jax.random.key(42), (E, K), minval=1e-6, maxval=1.0 - 1e-6))))


def _make_graphs_scatter(ch=2048):
    n_chunks = E // ch
    n_groups = ch // 16
    rounds = C // (16 * _NW)
    mesh = plsc.VectorSubcoreMesh(core_axis_name="c", subcore_axis_name="s")
    out_type = [jax.ShapeDtypeStruct((K, C, C), jnp.float32)]
    scratch = [
        pltpu.VMEM((16, C), jnp.float32),
        pltpu.VMEM((16, C), jnp.float32),
        pltpu.VMEM((16, C), jnp.int32),
        pltpu.VMEM((ch,), jnp.int32), pltpu.VMEM((ch,), jnp.int32),
        pltpu.VMEM((ch,), jnp.int32), pltpu.VMEM((ch,), jnp.int32),
        pltpu.VMEM((ch,), jnp.float32), pltpu.VMEM((ch,), jnp.float32),
        pltpu.VMEM((ch,), jnp.float32), pltpu.VMEM((ch,), jnp.float32),
        pltpu.SemaphoreType.DMA, pltpu.SemaphoreType.DMA,
    ]

    @functools.partial(pl.kernel, out_type=out_type, mesh=mesh,
                       scratch_types=scratch,
                       compiler_params=pltpu.CompilerParams(
                           use_tc_tiling_on_sc=False,
                           needs_layout_passes=False))
    def scatter_kernel(send, rec, e0, e1, graphs,
                       reg0, reg1, tmp, sa, sb, ra, rb_,
                       va0, vb0, va1, vb1, sem_a, sem_b):
        svs = (sa, sb)
        rvs = (ra, rb_)
        v0s = (va0, vb0)
        v1s = (va1, vb1)
        sems = (sem_a, sem_b)
        wid = lax.axis_index("s") * _NC + lax.axis_index("c")
        lane = lax.iota(jnp.int32, 16)
        zero16 = jnp.zeros((16,), jnp.float32)

        def fire(c_idx, b):
            off = c_idx * ch
            pltpu.async_copy(send.at[pl.ds(off, ch)], svs[b], sems[b])
            pltpu.async_copy(rec.at[pl.ds(off, ch)], rvs[b], sems[b])
            pltpu.async_copy(e0.at[pl.ds(off, ch)], v0s[b], sems[b])
            pltpu.async_copy(e1.at[pl.ds(off, ch)], v1s[b], sems[b])

        def drain(b):
            pltpu.make_async_copy(send.at[pl.ds(0, ch)], svs[b], sems[b]).wait()
            pltpu.make_async_copy(rec.at[pl.ds(0, ch)], rvs[b], sems[b]).wait()
            pltpu.make_async_copy(e0.at[pl.ds(0, ch)], v0s[b], sems[b]).wait()
            pltpu.make_async_copy(e1.at[pl.ds(0, ch)], v1s[b], sems[b]).wait()

        for r in range(rounds):
            lo = (r * _NW + wid) * 16

            @plsc.parallel_loop(0, 16 * C // 16, unroll=8)
            def zstep(j):
                row = j >> 7
                col = (j & 127) * 16
                reg0[row, pl.ds(col, 16)] = zero16
                reg1[row, pl.ds(col, 16)] = zero16

            def process(c_rel, b, lo=lo):
                def handle16(off16):
                    s = svs[b][pl.ds(off16, 16)]
                    valid = (s >= lo) & (s < lo + 16)

                    def dowork():
                        rr = rvs[b][pl.ds(off16, 16)]
                        val0 = v0s[b][pl.ds(off16, 16)]
                        val1 = v1s[b][pl.ds(off16, 16)]
                        rowv = jnp.clip(s - lo, 0, 15)
                        plsc.store_scatter(tmp, [rowv, rr], lane, mask=valid)
                        rb = plsc.load_gather(tmp, [rowv, rr], mask=valid)
                        anydup = jnp.any(valid & (rb != lane))

                        def fast():
                            ok = valid & (rb == lane)
                            plsc.store_scatter(reg0, [rowv, rr], val0, mask=ok)
                            plsc.store_scatter(reg1, [rowv, rr], val1, mask=ok)

                        def slow():
                            def sstep(j, carry2):
                                mj = valid & (lane == j)
                                plsc.store_scatter(reg0, [rowv, rr], val0, mask=mj)
                                plsc.store_scatter(reg1, [rowv, rr], val1, mask=mj)
                                return carry2
                            lax.fori_loop(0, 16, sstep, 0)

                        lax.cond(anydup, slow, fast)

                    lax.cond(jnp.any(valid), dowork, lambda: None)

                def gstep(g, carry):
                    sA = svs[b][pl.ds(g * 32, 16)]
                    sB = svs[b][pl.ds(g * 32 + 16, 16)]
                    hitA = jnp.any((sA >= lo) & (sA < lo + 16))
                    hitB = jnp.any((sB >= lo) & (sB < lo + 16))

                    def dopair():
                        handle16(g * 32)
                        handle16(g * 32 + 16)

                    lax.cond(hitA | hitB, dopair, lambda: None)
                    return carry

                lax.fori_loop(0, n_groups // 2, gstep, 0)

            fire(0, 0)

            def pairstep(t, carry):
                c0 = 2 * t
                fire(c0 + 1, 1)
                drain(0)
                process(c0, 0)
                fire(jnp.minimum(c0 + 2, n_chunks - 1), 0)
                drain(1)
                process(c0 + 1, 1)
                return carry

            lax.fori_loop(0, n_chunks // 2, pairstep, 0)
            drain(0)
            pltpu.sync_copy(reg0, graphs.at[0, pl.ds(lo, 16)])
            pltpu.sync_copy(reg1, graphs.at[1, pl.ds(lo, 16)])

    return scatter_kernel


_graphs_scatter = _make_graphs_scatter()


_HI = jax.lax.Precision.DEFAULT
_BE = 4096
_NB = E // _BE


def _dot(a, b):
    return jnp.dot(a, b, precision=_HI)


def _node_mlp_kernel(x_ref, w1_ref, b1_ref, w2_ref, b2_ref, g_ref, be_ref,
                     out_ref):
    x = jnp.maximum(_dot(x_ref[...], w1_ref[...]) + b1_ref[...][None, :], 0.0)
    x = jnp.maximum(_dot(x, w2_ref[...]) + b2_ref[...][None, :], 0.0)
    mean = jnp.mean(x, axis=0, keepdims=True)
    var = jnp.mean((x - mean) * (x - mean), axis=0, keepdims=True)
    xn = (x - mean) * jax.lax.rsqrt(var + 1e-5)
    out_ref[...] = xn * g_ref[...][None, :] + be_ref[...][None, :]


def _node_mlp(x, p, name):
    return pl.pallas_call(
        _node_mlp_kernel,
        out_shape=jax.ShapeDtypeStruct((C, H), jnp.float32),
    )(x, p[name + '_w1'], p[name + '_b1'], p[name + '_w2'], p[name + '_b2'],
      p[name + '_g'], p[name + '_be'])


def _pack_psum(y):
    s1 = jnp.sum(y, axis=0, keepdims=True)
    s2 = jnp.sum(y * y, axis=0, keepdims=True)
    ps = jnp.concatenate([s1, s2], axis=0)
    ps = jnp.concatenate([ps, jnp.zeros((6, H), jnp.float32)], axis=0)
    return jnp.concatenate([ps, jnp.zeros((8, 128 - H), jnp.float32)], axis=1)


def _edge_enc2_kernel(xs_ref, xr_ref, w1a_ref, w1b_ref, b1_ref, w2_ref,
                      b2_ref, y_ref, ps_ref):
    i = pl.program_id(0)
    h = jnp.maximum(_dot(xs_ref[...], w1a_ref[...]) +
                    _dot(xr_ref[...], w1b_ref[...]) + b1_ref[...][None, :], 0.0)
    y = jnp.maximum(_dot(h, w2_ref[...]) + b2_ref[...][None, :], 0.0)
    y_ref[...] = y

    @pl.when(i == 0)
    def _():
        ps_ref[...] = jnp.zeros_like(ps_ref)

    ps_ref[...] += _pack_psum(y)


def _edge_enc4_kernel(xs_ref, xr_ref, y2_ref, a2_ref, c2_ref, w1a_ref,
                      w1b_ref, w1c_ref, b1_ref, w2_ref, b2_ref, y_ref, ps_ref):
    i = pl.program_id(0)
    skip = y2_ref[...] * a2_ref[...][None, :] + c2_ref[...][None, :]
    h = jnp.maximum(_dot(xs_ref[...], w1a_ref[...]) +
                    _dot(xr_ref[...], w1b_ref[...]) +
                    _dot(skip, w1c_ref[...]) + b1_ref[...][None, :], 0.0)
    y = jnp.maximum(_dot(h, w2_ref[...]) + b2_ref[...][None, :], 0.0)
    y_ref[...] = y

    @pl.when(i == 0)
    def _():
        ps_ref[...] = jnp.zeros_like(ps_ref)

    ps_ref[...] += _pack_psum(y)


def _edge_logits_kernel(y4_ref, a4_ref, c4_ref, fcw_ref, fcb_ref, gn_ref,
                        edges_ref, prob_ref):
    x4 = y4_ref[...] * a4_ref[...][None, :] + c4_ref[...][None, :]
    logits = _dot(x4, fcw_ref[...]) + fcb_ref[...][None, :]
    m = jnp.max(logits, axis=-1, keepdims=True)
    ex = jnp.exp(logits - m)
    prob_ref[...] = ex / jnp.sum(ex, axis=-1, keepdims=True)
    gl = (logits + gn_ref[...]) / TAU
    m2 = jnp.max(gl, axis=-1, keepdims=True)
    ex2 = jnp.exp(gl - m2)
    edges_ref[...] = ex2 / jnp.sum(ex2, axis=-1, keepdims=True)


def _edge_msg_kernel(ds_ref, dr_ref, edges_ref,
                     w1a0_ref, w1b0_ref, b10_ref, w20_ref, b20_ref,
                     w1a1_ref, w1b1_ref, b11_ref, w21_ref, b21_ref, out_ref):
    ed = edges_ref[...]
    acc = None
    for i, (w1a, w1b, b1, w2, b2) in enumerate((
            (w1a0_ref, w1b0_ref, b10_ref, w20_ref, b20_ref),
            (w1a1_ref, w1b1_ref, b11_ref, w21_ref, b21_ref))):
        m = jnp.maximum(_dot(ds_ref[...], w1a[...]) +
                        _dot(dr_ref[...], w1b[...]) + b1[...][None, :], 0.0)
        m = jnp.maximum(_dot(m, w2[...]) + b2[...][None, :], 0.0)
        m = m * ed[:, i:i + 1]
        acc = m if acc is None else acc + m
    out_ref[...] = acc


def _head_kernel(agg_ref, w1_ref, b1_ref, w2_ref, b2_ref, out_ref):
    agg = agg_ref[...] * (1.0 / C)
    pred = jnp.maximum(_dot(agg, w1_ref[...]) + b1_ref[...][None, :], 0.0)
    out_ref[...] = _dot(pred, w2_ref[...]) + b2_ref[...][None, :]


def _eblock(ncol=H):
    return pl.BlockSpec((_BE, ncol), lambda i: (i, 0))


def _wspec(shape):
    nd = len(shape)
    return pl.BlockSpec(shape, lambda i: (0,) * nd)


def _bn_affine(ps, p, name):
    s1 = ps[0, :H]
    s2 = ps[1, :H]
    mean = s1 / E
    var = s2 / E - mean * mean
    a = p[name + '_g'] * jax.lax.rsqrt(var + 1e-5)
    c = p[name + '_be'] - mean * a
    return a, c


def kernel(data, params, send_idx, rec_idx):
    p = params
    x1 = _node_mlp(data, p, 'enc1')
    xs1, xr1, ds0, dr0 = _gather2(x1, data, send_idx, rec_idx)

    w1 = p['enc2_w1']
    y2, ps2 = pl.pallas_call(
        _edge_enc2_kernel,
        grid=(_NB,),
        in_specs=[_eblock(), _eblock(), _wspec((H, H)), _wspec((H, H)),
                  _wspec((H,)), _wspec((H, H)), _wspec((H,))],
        out_specs=[_eblock(), pl.BlockSpec((8, 128), lambda i: (0, 0))],
        out_shape=[jax.ShapeDtypeStruct((E, H), jnp.float32),
                   jax.ShapeDtypeStruct((8, 128), jnp.float32)],
    )(xs1, xr1, w1[:H], w1[H:], p['enc2_b1'], p['enc2_w2'], p['enc2_b2'])
    a2, c2 = _bn_affine(ps2, p, 'enc2')

    x2n = y2 * a2[None, :] + c2[None, :]
    z = jax.ops.segment_sum(x2n, rec_idx, num_segments=C)
    x3 = _node_mlp(z / C, p, 'enc3')
    xs3, xr3 = _gather1(x3, send_idx, rec_idx)

    w1 = p['enc4_w1']
    y4, ps4 = pl.pallas_call(
        _edge_enc4_kernel,
        grid=(_NB,),
        in_specs=[_eblock(), _eblock(), _eblock(), _wspec((H,)), _wspec((H,)),
                  _wspec((H, H)), _wspec((H, H)), _wspec((H, H)),
                  _wspec((H,)), _wspec((H, H)), _wspec((H,))],
        out_specs=[_eblock(), pl.BlockSpec((8, 128), lambda i: (0, 0))],
        out_shape=[jax.ShapeDtypeStruct((E, H), jnp.float32),
                   jax.ShapeDtypeStruct((8, 128), jnp.float32)],
    )(xs3, xr3, y2, a2, c2, w1[:H], w1[H:2 * H], w1[2 * H:],
      p['enc4_b1'], p['enc4_w2'], p['enc4_b2'])
    a4, c4 = _bn_affine(ps4, p, 'enc4')

    gn = jnp.asarray(_GN_NP)
    edges, prob = pl.pallas_call(
        _edge_logits_kernel,
        grid=(_NB,),
        in_specs=[_eblock(), _wspec((H,)), _wspec((H,)), _wspec((H, K)),
                  _wspec((K,)), _eblock(K)],
        out_specs=[_eblock(K), _eblock(K)],
        out_shape=[jax.ShapeDtypeStruct((E, K), jnp.float32),
                   jax.ShapeDtypeStruct((E, K), jnp.float32)],
    )(y4, a4, c4, p['fc_out_w'], p['fc_out_b'], gn)

    all_msgs = pl.pallas_call(
        _edge_msg_kernel,
        grid=(_NB,),
        in_specs=[_eblock(), _eblock(), _eblock(K),
                  _wspec((D, MSG_H)), _wspec((D, MSG_H)), _wspec((MSG_H,)),
                  _wspec((MSG_H, MSG_O)), _wspec((MSG_O,)),
                  _wspec((D, MSG_H)), _wspec((D, MSG_H)), _wspec((MSG_H,)),
                  _wspec((MSG_H, MSG_O)), _wspec((MSG_O,))],
        out_specs=_eblock(MSG_O),
        out_shape=jax.ShapeDtypeStruct((E, MSG_O), jnp.float32),
    )(ds0, dr0, edges,
      p['msg1_0_w'][:D], p['msg1_0_w'][D:], p['msg1_0_b'],
      p['msg2_0_w'], p['msg2_0_b'],
      p['msg1_1_w'][:D], p['msg1_1_w'][D:], p['msg1_1_b'],
      p['msg2_1_w'], p['msg2_1_b'])

    agg_raw = jax.ops.segment_sum(all_msgs, rec_idx, num_segments=C)
    output = pl.pallas_call(
        _head_kernel,
        out_shape=jax.ShapeDtypeStruct((C, D), jnp.float32),
    )(agg_raw, p['out1_w'], p['out1_b'], p['out2_w'], p['out2_b'])

    e0 = edges[:, 0] + 0.0
    e1 = edges[:, 1] + 0.0
    graphs = _graphs_scatter(send_idx, rec_idx, e0, e1)[0]
    return graphs, output, prob


# speedup vs baseline: 1.2072x; 1.0218x over previous
import functools

import jax
import jax.numpy as jnp
import numpy as np
from jax import lax
from jax.experimental import pallas as pl
from jax.experimental.pallas import tpu as pltpu
from jax.experimental.pallas import tpu_sc as plsc

C = 2048
E = 131072
D = 32
H = 32
K = 2
MSG_H = 64
MSG_O = 32
TAU = 0.1

_NC = 2
_NS = 16
_NW = _NC * _NS


def _make_gather(num_tables, chunk=1024):
    per_w = E // _NW
    n_chunks = per_w // chunk
    n_groups = chunk // 16
    mesh = plsc.VectorSubcoreMesh(core_axis_name="c", subcore_axis_name="s")
    out_type = [jax.ShapeDtypeStruct((E, D), jnp.float32)] * (2 * num_tables)
    scratch = [
        pltpu.VMEM((C, D), jnp.float32),
        pltpu.VMEM((chunk,), jnp.int32),
        pltpu.VMEM((chunk,), jnp.int32),
        pltpu.VMEM((chunk, D), jnp.float32),
    ]

    @functools.partial(pl.kernel, out_type=out_type, mesh=mesh,
                       scratch_types=scratch,
                       compiler_params=pltpu.CompilerParams(
                           use_tc_tiling_on_sc=False,
                           needs_layout_passes=False))
    def gather_kernel(*refs):
        tables = refs[:num_tables]
        send, rec = refs[num_tables], refs[num_tables + 1]
        outs = refs[num_tables + 2:3 * num_tables + 2]
        table_v, sidx, ridx, obuf = refs[3 * num_tables + 2:3 * num_tables + 6]
        wid = lax.axis_index("s") * _NC + lax.axis_index("c")
        base = wid * per_w
        lane = lax.iota(jnp.int32, 16)

        for ti in range(num_tables):
            pltpu.sync_copy(tables[ti], table_v)

            def chunk_step(t, carry, ti=ti):
                off = base + t * chunk
                pltpu.sync_copy(send.at[pl.ds(off, chunk)], sidx)
                pltpu.sync_copy(rec.at[pl.ds(off, chunk)], ridx)
                for which, idx_ref in ((0, sidx), (1, ridx)):
                    @plsc.parallel_loop(0, n_groups, unroll=8)
                    def group_step(g, idx_ref=idx_ref):
                        rows = idx_ref[pl.ds(g * 16, 16)]
                        orow = lane + g * 16
                        for j in range(D):
                            jcol = jnp.full((16,), j, jnp.int32)
                            vals = plsc.load_gather(table_v, [rows, jcol])
                            plsc.store_scatter(obuf, [orow, jcol], vals)
                    pltpu.sync_copy(obuf, outs[2 * ti + which].at[pl.ds(off, chunk)])
                return carry

            lax.fori_loop(0, n_chunks, chunk_step, 0)

    return gather_kernel


_gather2 = _make_gather(2)
_gather1 = _make_gather(1)

_GN_NP = np.asarray(
    -jnp.log(-jnp.log(jax.random.uniform(
        jax.random.key(42), (E, K), minval=1e-6, maxval=1.0 - 1e-6))))


def _make_graphs_scatter(ch=2048):
    n_chunks = E // ch
    n_groups = ch // 16
    rounds = C // (16 * _NW)
    mesh = plsc.VectorSubcoreMesh(core_axis_name="c", subcore_axis_name="s")
    out_type = [jax.ShapeDtypeStruct((K, C, C), jnp.float32)]
    scratch = [
        pltpu.VMEM((16, C), jnp.float32),
        pltpu.VMEM((16, C), jnp.float32),
        pltpu.VMEM((16, C), jnp.int32),
        pltpu.VMEM((ch,), jnp.int32), pltpu.VMEM((ch,), jnp.int32),
        pltpu.VMEM((ch,), jnp.int32), pltpu.VMEM((ch,), jnp.int32),
        pltpu.VMEM((ch,), jnp.float32), pltpu.VMEM((ch,), jnp.float32),
        pltpu.VMEM((ch,), jnp.float32), pltpu.VMEM((ch,), jnp.float32),
        pltpu.SemaphoreType.DMA, pltpu.SemaphoreType.DMA,
    ]

    @functools.partial(pl.kernel, out_type=out_type, mesh=mesh,
                       scratch_types=scratch,
                       compiler_params=pltpu.CompilerParams(
                           use_tc_tiling_on_sc=False,
                           needs_layout_passes=False))
    def scatter_kernel(send, rec, e0, e1, graphs,
                       reg0, reg1, tmp, sa, sb, ra, rb_,
                       va0, vb0, va1, vb1, sem_a, sem_b):
        svs = (sa, sb)
        rvs = (ra, rb_)
        v0s = (va0, vb0)
        v1s = (va1, vb1)
        sems = (sem_a, sem_b)
        wid = lax.axis_index("s") * _NC + lax.axis_index("c")
        lane = lax.iota(jnp.int32, 16)
        zero16 = jnp.zeros((16,), jnp.float32)

        def fire(c_idx, b):
            off = c_idx * ch
            pltpu.async_copy(send.at[pl.ds(off, ch)], svs[b], sems[b])
            pltpu.async_copy(rec.at[pl.ds(off, ch)], rvs[b], sems[b])
            pltpu.async_copy(e0.at[pl.ds(off, ch)], v0s[b], sems[b])
            pltpu.async_copy(e1.at[pl.ds(off, ch)], v1s[b], sems[b])

        def drain(b):
            pltpu.make_async_copy(send.at[pl.ds(0, ch)], svs[b], sems[b]).wait()
            pltpu.make_async_copy(rec.at[pl.ds(0, ch)], rvs[b], sems[b]).wait()
            pltpu.make_async_copy(e0.at[pl.ds(0, ch)], v0s[b], sems[b]).wait()
            pltpu.make_async_copy(e1.at[pl.ds(0, ch)], v1s[b], sems[b]).wait()

        for r in range(rounds):
            lo = (r * _NW + wid) * 16

            @plsc.parallel_loop(0, 16 * C // 16, unroll=8)
            def zstep(j):
                row = j >> 7
                col = (j & 127) * 16
                reg0[row, pl.ds(col, 16)] = zero16
                reg1[row, pl.ds(col, 16)] = zero16

            def process(c_rel, b, lo=lo):
                def handle16(off16):
                    s = svs[b][pl.ds(off16, 16)]
                    valid = (s >= lo) & (s < lo + 16)

                    def dowork():
                        rr = rvs[b][pl.ds(off16, 16)]
                        val0 = v0s[b][pl.ds(off16, 16)]
                        val1 = v1s[b][pl.ds(off16, 16)]
                        rowv = jnp.clip(s - lo, 0, 15)
                        plsc.store_scatter(tmp, [rowv, rr], lane, mask=valid)
                        rb = plsc.load_gather(tmp, [rowv, rr], mask=valid)
                        anydup = jnp.any(valid & (rb != lane))

                        def fast():
                            ok = valid & (rb == lane)
                            plsc.store_scatter(reg0, [rowv, rr], val0, mask=ok)
                            plsc.store_scatter(reg1, [rowv, rr], val1, mask=ok)

                        def slow():
                            def sstep(j, carry2):
                                mj = valid & (lane == j)
                                plsc.store_scatter(reg0, [rowv, rr], val0, mask=mj)
                                plsc.store_scatter(reg1, [rowv, rr], val1, mask=mj)
                                return carry2
                            lax.fori_loop(0, 16, sstep, 0)

                        lax.cond(anydup, slow, fast)

                    lax.cond(jnp.any(valid), dowork, lambda: None)

                def gstep(g, carry):
                    hit = None
                    for q in range(4):
                        sq = svs[b][pl.ds(g * 64 + q * 16, 16)]
                        hq = jnp.any((sq >= lo) & (sq < lo + 16))
                        hit = hq if hit is None else hit | hq

                    def doquad():
                        for q in range(4):
                            handle16(g * 64 + q * 16)

                    lax.cond(hit, doquad, lambda: None)
                    return carry

                lax.fori_loop(0, n_groups // 4, gstep, 0)

            fire(0, 0)

            def pairstep(t, carry):
                c0 = 2 * t
                fire(c0 + 1, 1)
                drain(0)
                process(c0, 0)
                fire(jnp.minimum(c0 + 2, n_chunks - 1), 0)
                drain(1)
                process(c0 + 1, 1)
                return carry

            lax.fori_loop(0, n_chunks // 2, pairstep, 0)
            drain(0)
            pltpu.sync_copy(reg0, graphs.at[0, pl.ds(lo, 16)])
            pltpu.sync_copy(reg1, graphs.at[1, pl.ds(lo, 16)])

    return scatter_kernel


_graphs_scatter = _make_graphs_scatter()


_HI = jax.lax.Precision.DEFAULT
_BE = 4096
_NB = E // _BE


def _dot(a, b):
    return jnp.dot(a, b, precision=_HI)


def _node_mlp_kernel(x_ref, w1_ref, b1_ref, w2_ref, b2_ref, g_ref, be_ref,
                     out_ref):
    x = jnp.maximum(_dot(x_ref[...], w1_ref[...]) + b1_ref[...][None, :], 0.0)
    x = jnp.maximum(_dot(x, w2_ref[...]) + b2_ref[...][None, :], 0.0)
    mean = jnp.mean(x, axis=0, keepdims=True)
    var = jnp.mean((x - mean) * (x - mean), axis=0, keepdims=True)
    xn = (x - mean) * jax.lax.rsqrt(var + 1e-5)
    out_ref[...] = xn * g_ref[...][None, :] + be_ref[...][None, :]


def _node_mlp(x, p, name):
    return pl.pallas_call(
        _node_mlp_kernel,
        out_shape=jax.ShapeDtypeStruct((C, H), jnp.float32),
    )(x, p[name + '_w1'], p[name + '_b1'], p[name + '_w2'], p[name + '_b2'],
      p[name + '_g'], p[name + '_be'])


def _pack_psum(y):
    s1 = jnp.sum(y, axis=0, keepdims=True)
    s2 = jnp.sum(y * y, axis=0, keepdims=True)
    ps = jnp.concatenate([s1, s2], axis=0)
    ps = jnp.concatenate([ps, jnp.zeros((6, H), jnp.float32)], axis=0)
    return jnp.concatenate([ps, jnp.zeros((8, 128 - H), jnp.float32)], axis=1)


def _edge_enc2_kernel(xs_ref, xr_ref, w1a_ref, w1b_ref, b1_ref, w2_ref,
                      b2_ref, y_ref, ps_ref):
    i = pl.program_id(0)
    h = jnp.maximum(_dot(xs_ref[...], w1a_ref[...]) +
                    _dot(xr_ref[...], w1b_ref[...]) + b1_ref[...][None, :], 0.0)
    y = jnp.maximum(_dot(h, w2_ref[...]) + b2_ref[...][None, :], 0.0)
    y_ref[...] = y

    @pl.when(i == 0)
    def _():
        ps_ref[...] = jnp.zeros_like(ps_ref)

    ps_ref[...] += _pack_psum(y)


def _edge_enc4_kernel(xs_ref, xr_ref, y2_ref, a2_ref, c2_ref, w1a_ref,
                      w1b_ref, w1c_ref, b1_ref, w2_ref, b2_ref, y_ref, ps_ref):
    i = pl.program_id(0)
    skip = y2_ref[...] * a2_ref[...][None, :] + c2_ref[...][None, :]
    h = jnp.maximum(_dot(xs_ref[...], w1a_ref[...]) +
                    _dot(xr_ref[...], w1b_ref[...]) +
                    _dot(skip, w1c_ref[...]) + b1_ref[...][None, :], 0.0)
    y = jnp.maximum(_dot(h, w2_ref[...]) + b2_ref[...][None, :], 0.0)
    y_ref[...] = y

    @pl.when(i == 0)
    def _():
        ps_ref[...] = jnp.zeros_like(ps_ref)

    ps_ref[...] += _pack_psum(y)


def _edge_logits_kernel(y4_ref, a4_ref, c4_ref, fcw_ref, fcb_ref, gn_ref,
                        edges_ref, prob_ref):
    x4 = y4_ref[...] * a4_ref[...][None, :] + c4_ref[...][None, :]
    logits = _dot(x4, fcw_ref[...]) + fcb_ref[...][None, :]
    m = jnp.max(logits, axis=-1, keepdims=True)
    ex = jnp.exp(logits - m)
    prob_ref[...] = ex / jnp.sum(ex, axis=-1, keepdims=True)
    gl = (logits + gn_ref[...]) / TAU
    m2 = jnp.max(gl, axis=-1, keepdims=True)
    ex2 = jnp.exp(gl - m2)
    edges_ref[...] = ex2 / jnp.sum(ex2, axis=-1, keepdims=True)


def _edge_msg_kernel(ds_ref, dr_ref, edges_ref,
                     w1a0_ref, w1b0_ref, b10_ref, w20_ref, b20_ref,
                     w1a1_ref, w1b1_ref, b11_ref, w21_ref, b21_ref, out_ref):
    ed = edges_ref[...]
    acc = None
    for i, (w1a, w1b, b1, w2, b2) in enumerate((
            (w1a0_ref, w1b0_ref, b10_ref, w20_ref, b20_ref),
            (w1a1_ref, w1b1_ref, b11_ref, w21_ref, b21_ref))):
        m = jnp.maximum(_dot(ds_ref[...], w1a[...]) +
                        _dot(dr_ref[...], w1b[...]) + b1[...][None, :], 0.0)
        m = jnp.maximum(_dot(m, w2[...]) + b2[...][None, :], 0.0)
        m = m * ed[:, i:i + 1]
        acc = m if acc is None else acc + m
    out_ref[...] = acc


def _head_kernel(agg_ref, w1_ref, b1_ref, w2_ref, b2_ref, out_ref):
    agg = agg_ref[...] * (1.0 / C)
    pred = jnp.maximum(_dot(agg, w1_ref[...]) + b1_ref[...][None, :], 0.0)
    out_ref[...] = _dot(pred, w2_ref[...]) + b2_ref[...][None, :]


def _eblock(ncol=H):
    return pl.BlockSpec((_BE, ncol), lambda i: (i, 0))


def _wspec(shape):
    nd = len(shape)
    return pl.BlockSpec(shape, lambda i: (0,) * nd)


def _bn_affine(ps, p, name):
    s1 = ps[0, :H]
    s2 = ps[1, :H]
    mean = s1 / E
    var = s2 / E - mean * mean
    a = p[name + '_g'] * jax.lax.rsqrt(var + 1e-5)
    c = p[name + '_be'] - mean * a
    return a, c


def kernel(data, params, send_idx, rec_idx):
    p = params
    ds0, dr0 = _gather1(data, send_idx, rec_idx)
    x1 = _node_mlp(data, p, 'enc1')
    xs1, xr1 = _gather1(x1, send_idx, rec_idx)

    w1 = p['enc2_w1']
    y2, ps2 = pl.pallas_call(
        _edge_enc2_kernel,
        grid=(_NB,),
        in_specs=[_eblock(), _eblock(), _wspec((H, H)), _wspec((H, H)),
                  _wspec((H,)), _wspec((H, H)), _wspec((H,))],
        out_specs=[_eblock(), pl.BlockSpec((8, 128), lambda i: (0, 0))],
        out_shape=[jax.ShapeDtypeStruct((E, H), jnp.float32),
                   jax.ShapeDtypeStruct((8, 128), jnp.float32)],
    )(xs1, xr1, w1[:H], w1[H:], p['enc2_b1'], p['enc2_w2'], p['enc2_b2'])
    a2, c2 = _bn_affine(ps2, p, 'enc2')

    x2n = y2 * a2[None, :] + c2[None, :]
    z = jax.ops.segment_sum(x2n, rec_idx, num_segments=C)
    x3 = _node_mlp(z / C, p, 'enc3')
    xs3, xr3 = _gather1(x3, send_idx, rec_idx)

    w1 = p['enc4_w1']
    y4, ps4 = pl.pallas_call(
        _edge_enc4_kernel,
        grid=(_NB,),
        in_specs=[_eblock(), _eblock(), _eblock(), _wspec((H,)), _wspec((H,)),
                  _wspec((H, H)), _wspec((H, H)), _wspec((H, H)),
                  _wspec((H,)), _wspec((H, H)), _wspec((H,))],
        out_specs=[_eblock(), pl.BlockSpec((8, 128), lambda i: (0, 0))],
        out_shape=[jax.ShapeDtypeStruct((E, H), jnp.float32),
                   jax.ShapeDtypeStruct((8, 128), jnp.float32)],
    )(xs3, xr3, y2, a2, c2, w1[:H], w1[H:2 * H], w1[2 * H:],
      p['enc4_b1'], p['enc4_w2'], p['enc4_b2'])
    a4, c4 = _bn_affine(ps4, p, 'enc4')

    gn = jnp.asarray(_GN_NP)
    edges, prob = pl.pallas_call(
        _edge_logits_kernel,
        grid=(_NB,),
        in_specs=[_eblock(), _wspec((H,)), _wspec((H,)), _wspec((H, K)),
                  _wspec((K,)), _eblock(K)],
        out_specs=[_eblock(K), _eblock(K)],
        out_shape=[jax.ShapeDtypeStruct((E, K), jnp.float32),
                   jax.ShapeDtypeStruct((E, K), jnp.float32)],
    )(y4, a4, c4, p['fc_out_w'], p['fc_out_b'], gn)

    all_msgs = pl.pallas_call(
        _edge_msg_kernel,
        grid=(_NB,),
        in_specs=[_eblock(), _eblock(), _eblock(K),
                  _wspec((D, MSG_H)), _wspec((D, MSG_H)), _wspec((MSG_H,)),
                  _wspec((MSG_H, MSG_O)), _wspec((MSG_O,)),
                  _wspec((D, MSG_H)), _wspec((D, MSG_H)), _wspec((MSG_H,)),
                  _wspec((MSG_H, MSG_O)), _wspec((MSG_O,))],
        out_specs=_eblock(MSG_O),
        out_shape=jax.ShapeDtypeStruct((E, MSG_O), jnp.float32),
    )(ds0, dr0, edges,
      p['msg1_0_w'][:D], p['msg1_0_w'][D:], p['msg1_0_b'],
      p['msg2_0_w'], p['msg2_0_b'],
      p['msg1_1_w'][:D], p['msg1_1_w'][D:], p['msg1_1_b'],
      p['msg2_1_w'], p['msg2_1_b'])

    agg_raw = jax.ops.segment_sum(all_msgs, rec_idx, num_segments=C)
    output = pl.pallas_call(
        _head_kernel,
        out_shape=jax.ShapeDtypeStruct((C, D), jnp.float32),
    )(agg_raw, p['out1_w'], p['out1_b'], p['out2_w'], p['out2_b'])

    e0 = edges[:, 0] + 0.0
    e1 = edges[:, 1] + 0.0
    graphs = _graphs_scatter(send_idx, rec_idx, e0, e1)[0]
    return graphs, output, prob


# speedup vs baseline: 1.2081x; 1.0007x over previous
import functools

import jax
import jax.numpy as jnp
import numpy as np
from jax import lax
from jax.experimental import pallas as pl
from jax.experimental.pallas import tpu as pltpu
from jax.experimental.pallas import tpu_sc as plsc

C = 2048
E = 131072
D = 32
H = 32
K = 2
MSG_H = 64
MSG_O = 32
TAU = 0.1

_NC = 2
_NS = 16
_NW = _NC * _NS


def _make_gather(num_tables, chunk=1024):
    per_w = E // _NW
    n_chunks = per_w // chunk
    n_groups = chunk // 16
    mesh = plsc.VectorSubcoreMesh(core_axis_name="c", subcore_axis_name="s")
    out_type = [jax.ShapeDtypeStruct((E, D), jnp.float32)] * (2 * num_tables)
    scratch = [
        pltpu.VMEM((C, D), jnp.float32),
        pltpu.VMEM((chunk,), jnp.int32),
        pltpu.VMEM((chunk,), jnp.int32),
        pltpu.VMEM((chunk, D), jnp.float32),
    ]

    @functools.partial(pl.kernel, out_type=out_type, mesh=mesh,
                       scratch_types=scratch,
                       compiler_params=pltpu.CompilerParams(
                           use_tc_tiling_on_sc=False,
                           needs_layout_passes=False))
    def gather_kernel(*refs):
        tables = refs[:num_tables]
        send, rec = refs[num_tables], refs[num_tables + 1]
        outs = refs[num_tables + 2:3 * num_tables + 2]
        table_v, sidx, ridx, obuf = refs[3 * num_tables + 2:3 * num_tables + 6]
        wid = lax.axis_index("s") * _NC + lax.axis_index("c")
        base = wid * per_w
        lane = lax.iota(jnp.int32, 16)

        for ti in range(num_tables):
            pltpu.sync_copy(tables[ti], table_v)

            def chunk_step(t, carry, ti=ti):
                off = base + t * chunk
                pltpu.sync_copy(send.at[pl.ds(off, chunk)], sidx)
                pltpu.sync_copy(rec.at[pl.ds(off, chunk)], ridx)
                for which, idx_ref in ((0, sidx), (1, ridx)):
                    @plsc.parallel_loop(0, n_groups, unroll=8)
                    def group_step(g, idx_ref=idx_ref):
                        rows = idx_ref[pl.ds(g * 16, 16)]
                        orow = lane + g * 16
                        for j in range(D):
                            jcol = jnp.full((16,), j, jnp.int32)
                            vals = plsc.load_gather(table_v, [rows, jcol])
                            plsc.store_scatter(obuf, [orow, jcol], vals)
                    pltpu.sync_copy(obuf, outs[2 * ti + which].at[pl.ds(off, chunk)])
                return carry

            lax.fori_loop(0, n_chunks, chunk_step, 0)

    return gather_kernel


_gather1 = _make_gather(1)

_GN_NP = np.asarray(
    -jnp.log(-jnp.log(jax.random.uniform(
        jax.random.key(42), (E, K), minval=1e-6, maxval=1.0 - 1e-6))))


def _make_graphs_scatter(ch=2048):
    n_chunks = E // ch
    n_groups = ch // 16
    rounds = C // (16 * _NW)
    mesh = plsc.VectorSubcoreMesh(core_axis_name="c", subcore_axis_name="s")
    out_type = [jax.ShapeDtypeStruct((K, C, C), jnp.float32)]
    scratch = [
        pltpu.VMEM((16, C), jnp.float32),
        pltpu.VMEM((16, C), jnp.float32),
        pltpu.VMEM((16, C), jnp.int32),
        pltpu.VMEM((ch,), jnp.int32), pltpu.VMEM((ch,), jnp.int32),
        pltpu.VMEM((ch,), jnp.int32), pltpu.VMEM((ch,), jnp.int32),
        pltpu.VMEM((ch,), jnp.float32), pltpu.VMEM((ch,), jnp.float32),
        pltpu.VMEM((ch,), jnp.float32), pltpu.VMEM((ch,), jnp.float32),
        pltpu.SemaphoreType.DMA, pltpu.SemaphoreType.DMA,
    ]

    @functools.partial(pl.kernel, out_type=out_type, mesh=mesh,
                       scratch_types=scratch,
                       compiler_params=pltpu.CompilerParams(
                           use_tc_tiling_on_sc=False,
                           needs_layout_passes=False))
    def scatter_kernel(send, rec, e0, e1, graphs,
                       reg0, reg1, tmp, sa, sb, ra, rb_,
                       va0, vb0, va1, vb1, sem_a, sem_b):
        svs = (sa, sb)
        rvs = (ra, rb_)
        v0s = (va0, vb0)
        v1s = (va1, vb1)
        sems = (sem_a, sem_b)
        wid = lax.axis_index("s") * _NC + lax.axis_index("c")
        lane = lax.iota(jnp.int32, 16)
        zero16 = jnp.zeros((16,), jnp.float32)

        def fire(c_idx, b):
            off = c_idx * ch
            pltpu.async_copy(send.at[pl.ds(off, ch)], svs[b], sems[b])
            pltpu.async_copy(rec.at[pl.ds(off, ch)], rvs[b], sems[b])
            pltpu.async_copy(e0.at[pl.ds(off, ch)], v0s[b], sems[b])
            pltpu.async_copy(e1.at[pl.ds(off, ch)], v1s[b], sems[b])

        def drain(b):
            pltpu.make_async_copy(send.at[pl.ds(0, ch)], svs[b], sems[b]).wait()
            pltpu.make_async_copy(rec.at[pl.ds(0, ch)], rvs[b], sems[b]).wait()
            pltpu.make_async_copy(e0.at[pl.ds(0, ch)], v0s[b], sems[b]).wait()
            pltpu.make_async_copy(e1.at[pl.ds(0, ch)], v1s[b], sems[b]).wait()

        for r in range(rounds):
            lo = (r * _NW + wid) * 16

            @plsc.parallel_loop(0, 16 * C // 16, unroll=8)
            def zstep(j):
                row = j >> 7
                col = (j & 127) * 16
                reg0[row, pl.ds(col, 16)] = zero16
                reg1[row, pl.ds(col, 16)] = zero16

            def process(c_rel, b, lo=lo):
                def handle16(off16):
                    s = svs[b][pl.ds(off16, 16)]
                    valid = (s >= lo) & (s < lo + 16)

                    def dowork():
                        rr = rvs[b][pl.ds(off16, 16)]
                        val0 = v0s[b][pl.ds(off16, 16)]
                        val1 = v1s[b][pl.ds(off16, 16)]
                        rowv = jnp.clip(s - lo, 0, 15)
                        plsc.store_scatter(tmp, [rowv, rr], lane, mask=valid)
                        rb = plsc.load_gather(tmp, [rowv, rr], mask=valid)
                        anydup = jnp.any(valid & (rb != lane))

                        def fast():
                            ok = valid & (rb == lane)
                            plsc.store_scatter(reg0, [rowv, rr], val0, mask=ok)
                            plsc.store_scatter(reg1, [rowv, rr], val1, mask=ok)

                        def slow():
                            def sstep(j, carry2):
                                mj = valid & (lane == j)
                                plsc.store_scatter(reg0, [rowv, rr], val0, mask=mj)
                                plsc.store_scatter(reg1, [rowv, rr], val1, mask=mj)
                                return carry2
                            lax.fori_loop(0, 16, sstep, 0)

                        lax.cond(anydup, slow, fast)

                    lax.cond(jnp.any(valid), dowork, lambda: None)

                def gstep(g, carry):
                    hit = None
                    for q in range(4):
                        sq = svs[b][pl.ds(g * 64 + q * 16, 16)]
                        hq = jnp.any((sq >= lo) & (sq < lo + 16))
                        hit = hq if hit is None else hit | hq

                    def doquad():
                        for q in range(4):
                            handle16(g * 64 + q * 16)

                    lax.cond(hit, doquad, lambda: None)
                    return carry

                lax.fori_loop(0, n_groups // 4, gstep, 0)

            fire(0, 0)

            def pairstep(t, carry):
                c0 = 2 * t
                fire(c0 + 1, 1)
                drain(0)
                process(c0, 0)
                fire(jnp.minimum(c0 + 2, n_chunks - 1), 0)
                drain(1)
                process(c0 + 1, 1)
                return carry

            lax.fori_loop(0, n_chunks // 2, pairstep, 0)
            drain(0)
            pltpu.sync_copy(reg0, graphs.at[0, pl.ds(lo, 16)])
            pltpu.sync_copy(reg1, graphs.at[1, pl.ds(lo, 16)])

    return scatter_kernel


_graphs_scatter = _make_graphs_scatter()


_HI = jax.lax.Precision.DEFAULT
_BE = 4096
_NB = E // _BE


def _dot(a, b):
    return jnp.dot(a, b, precision=_HI)


def _node_mlp_kernel(x_ref, w1_ref, b1_ref, w2_ref, b2_ref, g_ref, be_ref,
                     out_ref):
    x = jnp.maximum(_dot(x_ref[...], w1_ref[...]) + b1_ref[...][None, :], 0.0)
    x = jnp.maximum(_dot(x, w2_ref[...]) + b2_ref[...][None, :], 0.0)
    mean = jnp.mean(x, axis=0, keepdims=True)
    var = jnp.mean((x - mean) * (x - mean), axis=0, keepdims=True)
    xn = (x - mean) * jax.lax.rsqrt(var + 1e-5)
    out_ref[...] = xn * g_ref[...][None, :] + be_ref[...][None, :]


def _node_mlp(x, p, name):
    return pl.pallas_call(
        _node_mlp_kernel,
        out_shape=jax.ShapeDtypeStruct((C, H), jnp.float32),
    )(x, p[name + '_w1'], p[name + '_b1'], p[name + '_w2'], p[name + '_b2'],
      p[name + '_g'], p[name + '_be'])


def _pack_psum(y):
    s1 = jnp.sum(y, axis=0, keepdims=True)
    s2 = jnp.sum(y * y, axis=0, keepdims=True)
    ps = jnp.concatenate([s1, s2], axis=0)
    ps = jnp.concatenate([ps, jnp.zeros((6, H), jnp.float32)], axis=0)
    return jnp.concatenate([ps, jnp.zeros((8, 128 - H), jnp.float32)], axis=1)


def _edge_enc2_kernel(xs_ref, xr_ref, w1a_ref, w1b_ref, b1_ref, w2_ref,
                      b2_ref, y_ref, ps_ref):
    i = pl.program_id(0)
    h = jnp.maximum(_dot(xs_ref[...], w1a_ref[...]) +
                    _dot(xr_ref[...], w1b_ref[...]) + b1_ref[...][None, :], 0.0)
    y = jnp.maximum(_dot(h, w2_ref[...]) + b2_ref[...][None, :], 0.0)
    y_ref[...] = y

    @pl.when(i == 0)
    def _():
        ps_ref[...] = jnp.zeros_like(ps_ref)

    ps_ref[...] += _pack_psum(y)


def _edge_enc4_kernel(xs_ref, xr_ref, y2_ref, a2_ref, c2_ref, w1a_ref,
                      w1b_ref, w1c_ref, b1_ref, w2_ref, b2_ref, y_ref, ps_ref):
    i = pl.program_id(0)
    skip = y2_ref[...] * a2_ref[...][None, :] + c2_ref[...][None, :]
    h = jnp.maximum(_dot(xs_ref[...], w1a_ref[...]) +
                    _dot(xr_ref[...], w1b_ref[...]) +
                    _dot(skip, w1c_ref[...]) + b1_ref[...][None, :], 0.0)
    y = jnp.maximum(_dot(h, w2_ref[...]) + b2_ref[...][None, :], 0.0)
    y_ref[...] = y

    @pl.when(i == 0)
    def _():
        ps_ref[...] = jnp.zeros_like(ps_ref)

    ps_ref[...] += _pack_psum(y)


def _edge_logits_kernel(y4_ref, a4_ref, c4_ref, fcw_ref, fcb_ref, gn_ref,
                        edges_ref, prob_ref):
    x4 = y4_ref[...] * a4_ref[...][None, :] + c4_ref[...][None, :]
    logits = _dot(x4, fcw_ref[...]) + fcb_ref[...][None, :]
    m = jnp.max(logits, axis=-1, keepdims=True)
    ex = jnp.exp(logits - m)
    prob_ref[...] = ex / jnp.sum(ex, axis=-1, keepdims=True)
    gl = (logits + gn_ref[...]) / TAU
    m2 = jnp.max(gl, axis=-1, keepdims=True)
    ex2 = jnp.exp(gl - m2)
    edges_ref[...] = ex2 / jnp.sum(ex2, axis=-1, keepdims=True)


def _edge_msg_kernel(ds_ref, dr_ref, edges_ref,
                     w1a0_ref, w1b0_ref, b10_ref, w20_ref, b20_ref,
                     w1a1_ref, w1b1_ref, b11_ref, w21_ref, b21_ref, out_ref):
    ed = edges_ref[...]
    acc = None
    for i, (w1a, w1b, b1, w2, b2) in enumerate((
            (w1a0_ref, w1b0_ref, b10_ref, w20_ref, b20_ref),
            (w1a1_ref, w1b1_ref, b11_ref, w21_ref, b21_ref))):
        m = jnp.maximum(_dot(ds_ref[...], w1a[...]) +
                        _dot(dr_ref[...], w1b[...]) + b1[...][None, :], 0.0)
        m = jnp.maximum(_dot(m, w2[...]) + b2[...][None, :], 0.0)
        m = m * ed[:, i:i + 1]
        acc = m if acc is None else acc + m
    out_ref[...] = acc


def _head_kernel(agg_ref, w1_ref, b1_ref, w2_ref, b2_ref, out_ref):
    agg = agg_ref[...] * (1.0 / C)
    pred = jnp.maximum(_dot(agg, w1_ref[...]) + b1_ref[...][None, :], 0.0)
    out_ref[...] = _dot(pred, w2_ref[...]) + b2_ref[...][None, :]


def _eblock(ncol=H):
    return pl.BlockSpec((_BE, ncol), lambda i: (i, 0))


def _wspec(shape):
    nd = len(shape)
    return pl.BlockSpec(shape, lambda i: (0,) * nd)


def _bn_affine(ps, p, name):
    s1 = ps[0, :H]
    s2 = ps[1, :H]
    mean = s1 / E
    var = s2 / E - mean * mean
    a = p[name + '_g'] * jax.lax.rsqrt(var + 1e-5)
    c = p[name + '_be'] - mean * a
    return a, c


def kernel(data, params, send_idx, rec_idx):
    p = params
    ds0, dr0 = _gather1(data, send_idx, rec_idx)
    x1 = _node_mlp(data, p, 'enc1')
    xs1, xr1 = _gather1(x1, send_idx, rec_idx)

    w1 = p['enc2_w1']
    y2, ps2 = pl.pallas_call(
        _edge_enc2_kernel,
        grid=(_NB,),
        in_specs=[_eblock(), _eblock(), _wspec((H, H)), _wspec((H, H)),
                  _wspec((H,)), _wspec((H, H)), _wspec((H,))],
        out_specs=[_eblock(), pl.BlockSpec((8, 128), lambda i: (0, 0))],
        out_shape=[jax.ShapeDtypeStruct((E, H), jnp.float32),
                   jax.ShapeDtypeStruct((8, 128), jnp.float32)],
    )(xs1, xr1, w1[:H], w1[H:], p['enc2_b1'], p['enc2_w2'], p['enc2_b2'])
    a2, c2 = _bn_affine(ps2, p, 'enc2')

    x2n = y2 * a2[None, :] + c2[None, :]
    z = jax.ops.segment_sum(x2n, rec_idx, num_segments=C)
    x3 = _node_mlp(z / C, p, 'enc3')
    xs3, xr3 = _gather1(x3, send_idx, rec_idx)

    w1 = p['enc4_w1']
    y4, ps4 = pl.pallas_call(
        _edge_enc4_kernel,
        grid=(_NB,),
        in_specs=[_eblock(), _eblock(), _eblock(), _wspec((H,)), _wspec((H,)),
                  _wspec((H, H)), _wspec((H, H)), _wspec((H, H)),
                  _wspec((H,)), _wspec((H, H)), _wspec((H,))],
        out_specs=[_eblock(), pl.BlockSpec((8, 128), lambda i: (0, 0))],
        out_shape=[jax.ShapeDtypeStruct((E, H), jnp.float32),
                   jax.ShapeDtypeStruct((8, 128), jnp.float32)],
    )(xs3, xr3, y2, a2, c2, w1[:H], w1[H:2 * H], w1[2 * H:],
      p['enc4_b1'], p['enc4_w2'], p['enc4_b2'])
    a4, c4 = _bn_affine(ps4, p, 'enc4')

    gn = jnp.asarray(_GN_NP)
    edges, prob = pl.pallas_call(
        _edge_logits_kernel,
        grid=(_NB,),
        in_specs=[_eblock(), _wspec((H,)), _wspec((H,)), _wspec((H, K)),
                  _wspec((K,)), _eblock(K)],
        out_specs=[_eblock(K), _eblock(K)],
        out_shape=[jax.ShapeDtypeStruct((E, K), jnp.float32),
                   jax.ShapeDtypeStruct((E, K), jnp.float32)],
    )(y4, a4, c4, p['fc_out_w'], p['fc_out_b'], gn)

    all_msgs = pl.pallas_call(
        _edge_msg_kernel,
        grid=(_NB,),
        in_specs=[_eblock(), _eblock(), _eblock(K),
                  _wspec((D, MSG_H)), _wspec((D, MSG_H)), _wspec((MSG_H,)),
                  _wspec((MSG_H, MSG_O)), _wspec((MSG_O,)),
                  _wspec((D, MSG_H)), _wspec((D, MSG_H)), _wspec((MSG_H,)),
                  _wspec((MSG_H, MSG_O)), _wspec((MSG_O,))],
        out_specs=_eblock(MSG_O),
        out_shape=jax.ShapeDtypeStruct((E, MSG_O), jnp.float32),
    )(ds0, dr0, edges,
      p['msg1_0_w'][:D], p['msg1_0_w'][D:], p['msg1_0_b'],
      p['msg2_0_w'], p['msg2_0_b'],
      p['msg1_1_w'][:D], p['msg1_1_w'][D:], p['msg1_1_b'],
      p['msg2_1_w'], p['msg2_1_b'])

    agg_raw = jax.ops.segment_sum(all_msgs, rec_idx, num_segments=C)
    output = pl.pallas_call(
        _head_kernel,
        out_shape=jax.ShapeDtypeStruct((C, D), jnp.float32),
    )(agg_raw, p['out1_w'], p['out1_b'], p['out2_w'], p['out2_b'])

    e0 = edges[:, 0] + 0.0
    e1 = edges[:, 1] + 0.0
    graphs = _graphs_scatter(send_idx, rec_idx, e0, e1)[0]
    return graphs, output, prob
